# Initial kernel scaffold; baseline (speedup 1.0000x reference)
import jax
import jax.numpy as jnp
from jax.experimental import pallas as pl


def kernel(edge, edge_w, N, E, out_features, CUDA):
    raise NotImplementedError("write your pallas kernel here")


# speedup vs baseline: 6.8423x; 6.8423x over previous
import functools

import jax
import jax.numpy as jnp
from jax import lax
from jax.experimental import pallas as pl
from jax.experimental.pallas import tpu as pltpu
from jax.experimental.pallas import tpu_sc as plsc

N_NODES = 10000
N_PAD = 10112
OUT_F = 128
NC = 2
NS = 16
NW = NC * NS
CHUNK = 80


def _sc_partial_sums(seg3d, edge_w, zeros_init):
    e_total = edge_w.shape[0]
    per_tile = e_total // NW
    n_chunks = per_tile // CHUNK
    rows_per_tile = N_PAD // NS

    mesh = plsc.VectorSubcoreMesh(core_axis_name="c", subcore_axis_name="s")

    @functools.partial(
        pl.kernel,
        out_type=jax.ShapeDtypeStruct((NC, N_PAD, OUT_F), jnp.float32),
        mesh=mesh,
        scratch_types=[
            pltpu.VMEM_SHARED((N_PAD, OUT_F), jnp.float32),
            pltpu.VMEM((n_chunks, CHUNK), jnp.int32),
            pltpu.VMEM((CHUNK, OUT_F), jnp.float32),
            pltpu.VMEM((CHUNK, OUT_F), jnp.float32),
            pltpu.SemaphoreType.DMA,
            pltpu.SemaphoreType.DMA,
        ],
    )
    def grouped(seg_hbm, ew_hbm, z_hbm, out_hbm, acc, idx_all, buf0, buf1,
                sem0, sem1):
        c = lax.axis_index("c")
        s = lax.axis_index("s")
        wid = c * NS + s
        row0 = s * rows_per_tile

        pltpu.sync_copy(z_hbm, acc.at[pl.ds(row0, rows_per_tile)])
        pltpu.sync_copy(seg_hbm.at[wid], idx_all)
        plsc.subcore_barrier()

        base_e = wid * per_tile
        dummy0 = ew_hbm.at[pl.ds(base_e, CHUNK)]

        pltpu.async_copy(dummy0, buf0, sem0)

        def body(i, carry):
            j0 = 2 * i
            j1 = j0 + 1

            @pl.when(j1 < n_chunks)
            def _start1():
                pltpu.async_copy(
                    ew_hbm.at[pl.ds(base_e + j1 * CHUNK, CHUNK)], buf1, sem1)

            pltpu.make_async_copy(dummy0, buf0, sem0).wait()
            pltpu.sync_copy(buf0, acc.at[idx_all.at[j0]], add=True)

            @pl.when(j1 < n_chunks)
            def _half1():
                @pl.when(j1 + 1 < n_chunks)
                def _start0():
                    pltpu.async_copy(
                        ew_hbm.at[pl.ds(base_e + (j1 + 1) * CHUNK, CHUNK)],
                        buf0, sem0)

                pltpu.make_async_copy(dummy0, buf1, sem1).wait()
                pltpu.sync_copy(buf1, acc.at[idx_all.at[j1]], add=True)

            return carry

        lax.fori_loop(0, (n_chunks + 1) // 2, body, 0)
        plsc.subcore_barrier()

        pltpu.sync_copy(acc.at[pl.ds(row0, rows_per_tile)],
                        out_hbm.at[c, pl.ds(row0, rows_per_tile)])

    return grouped(seg3d, edge_w, zeros_init)


def _combine(partials):
    blk = N_NODES // 10

    def body(p_ref, o_ref):
        o_ref[...] = p_ref[0] + p_ref[1]

    return pl.pallas_call(
        body,
        grid=(10,),
        in_specs=[pl.BlockSpec((2, blk, OUT_F), lambda i: (0, i, 0))],
        out_specs=pl.BlockSpec((blk, OUT_F), lambda i: (i, 0)),
        out_shape=jax.ShapeDtypeStruct((N_NODES, OUT_F), jnp.float32),
    )(partials)


def kernel(edge, edge_w, N, E, out_features, CUDA):
    seg3d = edge[0].reshape(NW, -1, CHUNK)
    zeros_init = jnp.zeros((N_PAD // NS, OUT_F), jnp.float32)
    partials = _sc_partial_sums(seg3d, edge_w, zeros_init)
    return _combine(partials)


# speedup vs baseline: 7.3192x; 1.0697x over previous
import functools

import jax
import jax.numpy as jnp
from jax import lax
from jax.experimental import pallas as pl
from jax.experimental.pallas import tpu as pltpu
from jax.experimental.pallas import tpu_sc as plsc

N_NODES = 10000
N_PAD = 10112
OUT_F = 128
NC = 2
NS = 16
NW = NC * NS
WIN = 128
N_MAIN = 78
E_MAIN = NW * N_MAIN * WIN
N_WINS = N_MAIN + 1


def _build_seg(seg):
    main = seg[:E_MAIN].reshape(NW, N_MAIN, WIN)
    pad_ids = N_NODES + (jnp.arange(NW * WIN, dtype=jnp.int32)
                         % (N_PAD - N_NODES))
    tail = pad_ids.reshape(NW, 1, WIN)
    n_tail_tiles = (seg.shape[0] - E_MAIN) // WIN
    tail = tail.at[:n_tail_tiles].set(
        seg[E_MAIN:].reshape(n_tail_tiles, 1, WIN))
    return jnp.concatenate([main, tail], axis=1)


def _sc_partial_sums(seg3d, edge_w, zeros_init):
    rows_per_tile = N_PAD // NS
    mesh = plsc.VectorSubcoreMesh(core_axis_name="c", subcore_axis_name="s")

    @functools.partial(
        pl.kernel,
        out_type=jax.ShapeDtypeStruct((NC, N_PAD, OUT_F), jnp.float32),
        mesh=mesh,
        scratch_types=[
            pltpu.VMEM_SHARED((N_PAD, OUT_F), jnp.float32),
            pltpu.VMEM((N_WINS, WIN), jnp.int32),
            pltpu.VMEM((WIN, OUT_F), jnp.float32),
            pltpu.VMEM((WIN, OUT_F), jnp.float32),
            pltpu.SemaphoreType.DMA,
            pltpu.SemaphoreType.DMA,
        ],
    )
    def grouped(seg_hbm, ew_hbm, z_hbm, out_hbm, acc, idx_all, buf0, buf1,
                sem0, sem1):
        c = lax.axis_index("c")
        s = lax.axis_index("s")
        wid = c * NS + s
        row0 = s * rows_per_tile

        pltpu.sync_copy(z_hbm, acc.at[pl.ds(row0, rows_per_tile)])
        pltpu.sync_copy(seg_hbm.at[wid], idx_all)
        plsc.subcore_barrier()

        main0 = wid * (N_MAIN * WIN)
        tail0 = jnp.where(wid < NW // 8, E_MAIN + wid * WIN, 0)

        def wslice(j):
            base = jnp.where(j < N_MAIN, main0 + j * WIN, tail0)
            return ew_hbm.at[pl.ds(pl.multiple_of(base, WIN), WIN)]

        dummy0 = ew_hbm.at[pl.ds(main0, WIN)]
        pltpu.async_copy(dummy0, buf0, sem0)

        def body(i, carry):
            j0 = 2 * i
            j1 = j0 + 1

            @pl.when(j1 < N_WINS)
            def _start1():
                pltpu.async_copy(wslice(j1), buf1, sem1)

            pltpu.make_async_copy(dummy0, buf0, sem0).wait()
            pltpu.sync_copy(buf0, acc.at[idx_all.at[j0]], add=True)

            @pl.when(j1 < N_WINS)
            def _half1():
                @pl.when(j1 + 1 < N_WINS)
                def _start0():
                    pltpu.async_copy(wslice(j1 + 1), buf0, sem0)

                pltpu.make_async_copy(dummy0, buf1, sem1).wait()
                pltpu.sync_copy(buf1, acc.at[idx_all.at[j1]], add=True)

            return carry

        lax.fori_loop(0, (N_WINS + 1) // 2, body, 0)
        plsc.subcore_barrier()

        pltpu.sync_copy(acc.at[pl.ds(row0, rows_per_tile)],
                        out_hbm.at[c, pl.ds(row0, rows_per_tile)])

    return grouped(seg3d, edge_w, zeros_init)


def _combine(partials):
    blk = N_NODES // 10

    def body(p_ref, o_ref):
        o_ref[...] = p_ref[0] + p_ref[1]

    return pl.pallas_call(
        body,
        grid=(10,),
        in_specs=[pl.BlockSpec((2, blk, OUT_F), lambda i: (0, i, 0))],
        out_specs=pl.BlockSpec((blk, OUT_F), lambda i: (i, 0)),
        out_shape=jax.ShapeDtypeStruct((N_NODES, OUT_F), jnp.float32),
    )(partials)


def kernel(edge, edge_w, N, E, out_features, CUDA):
    seg3d = _build_seg(edge[0])
    zeros_init = jnp.zeros((N_PAD // NS, OUT_F), jnp.float32)
    partials = _sc_partial_sums(seg3d, edge_w, zeros_init)
    return _combine(partials)


# speedup vs baseline: 9.4443x; 1.2903x over previous
import functools

import jax
import jax.numpy as jnp
from jax import lax
from jax.experimental import pallas as pl
from jax.experimental.pallas import tpu as pltpu
from jax.experimental.pallas import tpu_sc as plsc

N_NODES = 10000
N_PAD = 10112
OUT_F = 128
NC = 2
NS = 16
NW = NC * NS
WIN = 128
N_MAIN = 78
E_MAIN = NW * N_MAIN * WIN
LANES = 16
NBUF = 3


def _sc_partial_sums(edge, edge_w):
    rows_per_tile = N_PAD // NS
    n_tail_tiles = (edge_w.shape[0] - E_MAIN) // WIN
    mesh = plsc.VectorSubcoreMesh(core_axis_name="c", subcore_axis_name="s")

    @functools.partial(
        pl.kernel,
        out_type=jax.ShapeDtypeStruct((NC, N_PAD, OUT_F), jnp.float32),
        mesh=mesh,
        scratch_types=[
            pltpu.VMEM_SHARED((N_PAD, OUT_F), jnp.float32),
            pltpu.VMEM((WIN, OUT_F), jnp.float32),
            pltpu.VMEM((WIN, OUT_F), jnp.float32),
            pltpu.VMEM((WIN, OUT_F), jnp.float32),
            pltpu.VMEM((WIN,), jnp.int32),
            pltpu.VMEM((WIN,), jnp.int32),
            pltpu.VMEM((WIN,), jnp.int32),
            pltpu.SemaphoreType.DMA,
            pltpu.SemaphoreType.DMA,
            pltpu.SemaphoreType.DMA,
            pltpu.SemaphoreType.DMA,
        ],
    )
    def grouped(edge_hbm, ew_hbm, out_hbm, acc, buf0, buf1, buf2,
                idx0, idx1, idx2, sem0, sem1, sem2, zsem):
        c = lax.axis_index("c")
        s = lax.axis_index("s")
        wid = c * NS + s
        row0 = s * rows_per_tile
        main0 = wid * (N_MAIN * WIN)
        has_tail = wid < n_tail_tiles
        n_wins = N_MAIN + has_tail.astype(jnp.int32)

        zvec = jnp.zeros((LANES,), jnp.float32)

        def zrow(r, carry):
            for l in range(OUT_F // LANES):
                buf0[r, pl.ds(pl.multiple_of(l * LANES, LANES), LANES)] = zvec
            return carry

        lax.fori_loop(0, WIN, zrow, 0)
        n_zcopies = rows_per_tile // WIN
        z_rem = rows_per_tile - n_zcopies * WIN
        for k in range(n_zcopies):
            pltpu.async_copy(buf0, acc.at[pl.ds(row0 + k * WIN, WIN)], zsem)
        pltpu.async_copy(buf0.at[pl.ds(0, z_rem)],
                         acc.at[pl.ds(row0 + n_zcopies * WIN, z_rem)], zsem)
        for k in range(n_zcopies):
            pltpu.make_async_copy(buf0, acc.at[pl.ds(row0 + k * WIN, WIN)],
                                  zsem).wait()
        pltpu.make_async_copy(buf0.at[pl.ds(0, z_rem)],
                              acc.at[pl.ds(row0 + n_zcopies * WIN, z_rem)],
                              zsem).wait()

        def wbase(j):
            return pl.multiple_of(
                jnp.where(j < N_MAIN, main0 + j * WIN, E_MAIN + wid * WIN),
                WIN)

        def fetch(j, buf, idxb, sem):
            base = wbase(j)
            pltpu.async_copy(edge_hbm.at[0, pl.ds(base, WIN)], idxb, sem)
            pltpu.async_copy(ew_hbm.at[pl.ds(base, WIN)], buf, sem)

        dummy_i = edge_hbm.at[0, pl.ds(main0, WIN)]
        dummy_w = ew_hbm.at[pl.ds(main0, WIN)]

        def fetch_wait(buf, idxb, sem):
            pltpu.make_async_copy(dummy_i, idxb, sem).wait()
            pltpu.make_async_copy(dummy_w, buf, sem).wait()

        slots = ((buf0, idx0, sem0), (buf1, idx1, sem1), (buf2, idx2, sem2))
        for b in range(NBUF):
            fetch(jnp.int32(b), *slots[b])

        plsc.subcore_barrier()

        def body(g, carry):
            for b in range(NBUF):
                j = NBUF * g + b
                buf, idxb, sem = slots[b]

                @pl.when(j < n_wins)
                def _win():
                    fetch_wait(buf, idxb, sem)
                    pltpu.sync_copy(buf, acc.at[idxb], add=True)

                    @pl.when(j + NBUF < n_wins)
                    def _refill():
                        fetch(j + NBUF, buf, idxb, sem)

            return carry

        lax.fori_loop(0, (N_MAIN + 1 + NBUF - 1) // NBUF, body, 0)
        plsc.subcore_barrier()

        pltpu.sync_copy(acc.at[pl.ds(row0, rows_per_tile)],
                        out_hbm.at[c, pl.ds(row0, rows_per_tile)])

    return grouped(edge, edge_w)


def _combine(partials):
    n_blk = 5
    blk = N_NODES // n_blk

    def body(p_ref, o_ref):
        o_ref[...] = p_ref[0] + p_ref[1]

    return pl.pallas_call(
        body,
        grid=(n_blk,),
        in_specs=[pl.BlockSpec((2, blk, OUT_F), lambda i: (0, i, 0))],
        out_specs=pl.BlockSpec((blk, OUT_F), lambda i: (i, 0)),
        out_shape=jax.ShapeDtypeStruct((N_NODES, OUT_F), jnp.float32),
    )(partials)


def kernel(edge, edge_w, N, E, out_features, CUDA):
    partials = _sc_partial_sums(edge, edge_w)
    return _combine(partials)


# speedup vs baseline: 9.5375x; 1.0099x over previous
import functools

import jax
import jax.numpy as jnp
from jax import lax
from jax.experimental import pallas as pl
from jax.experimental.pallas import tpu as pltpu
from jax.experimental.pallas import tpu_sc as plsc

N_NODES = 10000
N_PAD = 10112
OUT_F = 128
NC = 2
NS = 16
NW = NC * NS
WIN = 128
N_MAIN = 78
E_MAIN = NW * N_MAIN * WIN
LANES = 16
NBUF = 3


def _sc_partial_sums(edge, edge_w):
    rows_per_tile = N_PAD // NS
    n_tail_tiles = (edge_w.shape[0] - E_MAIN) // WIN
    mesh = plsc.VectorSubcoreMesh(core_axis_name="c", subcore_axis_name="s")

    @functools.partial(
        pl.kernel,
        out_type=jax.ShapeDtypeStruct((NC, N_PAD, OUT_F), jnp.float32),
        mesh=mesh,
        scratch_types=[
            pltpu.VMEM_SHARED((N_PAD, OUT_F), jnp.float32),
            pltpu.VMEM((WIN, OUT_F), jnp.float32),
            pltpu.VMEM((WIN, OUT_F), jnp.float32),
            pltpu.VMEM((WIN, OUT_F), jnp.float32),
            pltpu.VMEM((WIN,), jnp.int32),
            pltpu.VMEM((WIN,), jnp.int32),
            pltpu.VMEM((WIN,), jnp.int32),
            pltpu.SemaphoreType.DMA,
            pltpu.SemaphoreType.DMA,
            pltpu.SemaphoreType.DMA,
            pltpu.SemaphoreType.DMA,
        ],
    )
    def grouped(edge_hbm, ew_hbm, out_hbm, acc, buf0, buf1, buf2,
                idx0, idx1, idx2, sem0, sem1, sem2, zsem):
        c = lax.axis_index("c")
        s = lax.axis_index("s")
        wid = c * NS + s
        row0 = s * rows_per_tile
        main0 = wid * (N_MAIN * WIN)
        has_tail = wid < n_tail_tiles
        n_wins = N_MAIN + has_tail.astype(jnp.int32)

        def wbase(j):
            return pl.multiple_of(
                jnp.where(j < N_MAIN, main0 + j * WIN, E_MAIN + wid * WIN),
                WIN)

        def fetch(j, buf, idxb, sem):
            base = wbase(j)
            pltpu.async_copy(edge_hbm.at[0, pl.ds(base, WIN)], idxb, sem)
            pltpu.async_copy(ew_hbm.at[pl.ds(base, WIN)], buf, sem)

        slots = ((buf0, idx0, sem0), (buf1, idx1, sem1), (buf2, idx2, sem2))
        for b in range(1, NBUF):
            fetch(jnp.int32(b), *slots[b])

        zvec = jnp.zeros((LANES,), jnp.float32)

        def zrow(r, carry):
            for l in range(OUT_F // LANES):
                buf0[r, pl.ds(pl.multiple_of(l * LANES, LANES), LANES)] = zvec
            return carry

        lax.fori_loop(0, WIN, zrow, 0)
        n_zcopies = rows_per_tile // WIN
        z_rem = rows_per_tile - n_zcopies * WIN
        for k in range(n_zcopies):
            pltpu.async_copy(buf0, acc.at[pl.ds(row0 + k * WIN, WIN)], zsem)
        pltpu.async_copy(buf0.at[pl.ds(0, z_rem)],
                         acc.at[pl.ds(row0 + n_zcopies * WIN, z_rem)], zsem)
        for k in range(n_zcopies):
            pltpu.make_async_copy(buf0, acc.at[pl.ds(row0 + k * WIN, WIN)],
                                  zsem).wait()
        pltpu.make_async_copy(buf0.at[pl.ds(0, z_rem)],
                              acc.at[pl.ds(row0 + n_zcopies * WIN, z_rem)],
                              zsem).wait()

        dummy_i = edge_hbm.at[0, pl.ds(main0, WIN)]
        dummy_w = ew_hbm.at[pl.ds(main0, WIN)]

        def fetch_wait(buf, idxb, sem):
            pltpu.make_async_copy(dummy_i, idxb, sem).wait()
            pltpu.make_async_copy(dummy_w, buf, sem).wait()

        fetch(jnp.int32(0), *slots[0])

        plsc.subcore_barrier()

        def body(g, carry):
            for b in range(NBUF):
                j = NBUF * g + b
                buf, idxb, sem = slots[b]

                @pl.when(j < n_wins)
                def _win():
                    fetch_wait(buf, idxb, sem)
                    pltpu.sync_copy(buf, acc.at[idxb], add=True)

                    @pl.when(j + NBUF < n_wins)
                    def _refill():
                        fetch(j + NBUF, buf, idxb, sem)

            return carry

        lax.fori_loop(0, (N_MAIN + 1 + NBUF - 1) // NBUF, body, 0)
        plsc.subcore_barrier()

        pltpu.sync_copy(acc.at[pl.ds(row0, rows_per_tile)],
                        out_hbm.at[c, pl.ds(row0, rows_per_tile)])

    return grouped(edge, edge_w)


def _combine(partials):
    n_blk = 5
    blk = N_NODES // n_blk

    def body(p_ref, o_ref):
        o_ref[...] = p_ref[0] + p_ref[1]

    return pl.pallas_call(
        body,
        grid=(n_blk,),
        in_specs=[pl.BlockSpec((2, blk, OUT_F), lambda i: (0, i, 0))],
        out_specs=pl.BlockSpec((blk, OUT_F), lambda i: (i, 0)),
        out_shape=jax.ShapeDtypeStruct((N_NODES, OUT_F), jnp.float32),
    )(partials)


def kernel(edge, edge_w, N, E, out_features, CUDA):
    partials = _sc_partial_sums(edge, edge_w)
    return _combine(partials)
